# baseline (speedup 1.0000x reference)
import jax
import jax.numpy as jnp
from jax.experimental import pallas as pl
from jax.experimental.pallas import tpu as pltpu

_R, _C = 2048, 8192
_BLK = 128
_NB = _R // _BLK
_NBOUND = 16
_NREFINE = 2
_PB = _NREFINE + 1
_K = int(0.05 * _R * _C)


def _stage_body(pre_ref, fine_ref, tvb_ref, max_ref, state):
    b = pl.program_id(0)
    tv = fine_ref[...] - pre_ref[...]
    tvb_ref[...] = tv.astype(jnp.bfloat16)
    m = jnp.max(jnp.abs(tv))
    state[0] = jnp.where(b == 0, m, jnp.maximum(state[0], m))

    @pl.when(b == _NB - 1)
    def _emit():
        max_ref[...] = jnp.full((8, 128), state[0] * 1.01, jnp.float32)


def _select_body(pre_ref, tvb_ref, max_ref, out_ref, state, cnt):
    p = pl.program_id(0)
    b = pl.program_id(1)

    @pl.when(p < _NREFINE)
    def _refine_phase():
        lo = jnp.where(p == 0, 0.0, state[0])
        hi = jnp.where(p == 0, max_ref[0, 0], state[1])
        absb = jnp.abs(tvb_ref[...])
        width = (hi - lo) * (1.0 / _NBOUND)
        tbs = [
            (lo + width * j).astype(jnp.bfloat16) for j in range(_NBOUND)
        ]
        for j in range(_NBOUND):
            c = jnp.sum(absb > tbs[j], dtype=jnp.int32)
            cnt[j] = jnp.where(b == 0, c, cnt[j] + c)

        @pl.when(b == _NB - 1)
        def _select():
            jstar = jnp.int32(0)
            for j in range(1, _NBOUND):
                jstar = jnp.where(cnt[j] >= _K, jnp.int32(j), jstar)
            tb32 = [t.astype(jnp.float32) for t in tbs]
            new_lo = tb32[0]
            new_hi = hi
            for j in range(1, _NBOUND):
                sel = jstar >= j
                new_lo = jnp.where(sel, tb32[j], new_lo)
                new_hi = jnp.where(jstar >= j - 1, tb32[j], new_hi)
            new_hi = jnp.where(jstar == _NBOUND - 1, hi, new_hi)
            state[0] = new_lo
            state[1] = new_hi

    @pl.when(p == _PB - 1)
    def _apply_phase():
        t = state[0].astype(jnp.bfloat16)
        tvb = tvb_ref[...]
        keep = jnp.abs(tvb) > t
        out_ref[...] = pre_ref[...] + jnp.where(
            keep, tvb.astype(jnp.float32), 0.0
        )


@jax.jit
def kernel(pretensor, finetensor):
    blk_f32 = pl.BlockSpec((_BLK, _C), lambda b: (b, 0))
    tvb, maxarr = pl.pallas_call(
        _stage_body,
        grid=(_NB,),
        in_specs=[blk_f32, blk_f32],
        out_specs=[
            pl.BlockSpec((_BLK, _C), lambda b: (b, 0)),
            pl.BlockSpec((8, 128), lambda b: (0, 0)),
        ],
        out_shape=[
            jax.ShapeDtypeStruct((_R, _C), jnp.bfloat16),
            jax.ShapeDtypeStruct((8, 128), jnp.float32),
        ],
        scratch_shapes=[pltpu.SMEM((1,), jnp.float32)],
    )(pretensor, finetensor)

    pre_spec = pl.BlockSpec(
        (_BLK, _C), lambda p, b: (jnp.where(p == _PB - 1, b, 0), 0)
    )
    tvb_spec = pl.BlockSpec((_BLK, _C), lambda p, b: (b, 0))
    max_spec = pl.BlockSpec((8, 128), lambda p, b: (0, 0))
    out_spec = pl.BlockSpec(
        (_BLK, _C), lambda p, b: (jnp.where(p == _PB - 1, b, 0), 0)
    )
    return pl.pallas_call(
        _select_body,
        grid=(_PB, _NB),
        in_specs=[pre_spec, tvb_spec, max_spec],
        out_specs=out_spec,
        out_shape=jax.ShapeDtypeStruct((_R, _C), jnp.float32),
        scratch_shapes=[
            pltpu.SMEM((2,), jnp.float32),
            pltpu.SMEM((_NBOUND,), jnp.int32),
        ],
    )(pretensor, tvb, maxarr)


# speedup vs baseline: 1.0300x; 1.0300x over previous
import functools

import jax
import jax.numpy as jnp
from jax import lax
from jax.experimental import pallas as pl
from jax.experimental.pallas import tpu as pltpu
from jax.experimental.pallas import tpu_sc as plsc

_R, _C = 2048, 8192
_N = _R * _C
_K = int(0.05 * _N)
_NBUCKETS = 4096
_SHIFT = 19

_NC, _NS = 2, 16
_NW = _NC * _NS
_SCROW = 32768
_NROWS = _N // _SCROW
_ROWS_PER_W = _NROWS // _NW
_CHUNK = 8192
_CHUNKS_PER_ROW = _SCROW // _CHUNK
_VECS = _CHUNK // 16

_BLK = 128
_NB = _R // _BLK


def _stage_body(pre_ref, fine_ref, idx_ref):
    tv = fine_ref[...] - pre_ref[...]
    bits = lax.bitcast_convert_type(tv, jnp.int32)
    idx_ref[...] = lax.shift_right_logical(
        jnp.bitwise_and(bits, jnp.int32(0x7FFFFFFF)), _SHIFT
    )


def _hist_body(idx_hbm, out_hbm, b0, b1, hist1d, merged, shared, s0, s1):
    cid = lax.axis_index("c")
    sid = lax.axis_index("s")
    wid = sid * _NC + cid
    base_row = wid * _ROWS_PER_W

    laneoff = lax.iota(jnp.int32, 16) * _NBUCKETS
    ones = jnp.ones((16,), jnp.int32)
    zeros = jnp.zeros((16,), jnp.int32)

    def zbody(c, _):
        hist1d[pl.ds(c * 16, 16)] = zeros
        return 0
    lax.fori_loop(0, 16 * _NBUCKETS // 16, zbody, 0)

    bufs = [b0, b1]
    sems = [s0, s1]

    def start(r, slot):
        row = base_row + (r // _CHUNKS_PER_ROW)
        off = (r % _CHUNKS_PER_ROW) * _CHUNK
        return pltpu.async_copy(
            idx_hbm.at[row, pl.ds(off, _CHUNK)], bufs[slot], sems[slot]
        )

    def process(buf_ref):
        def body(i, _):
            bkt = buf_ref[pl.ds(i * 16, 16)]
            plsc.addupdate_scatter(hist1d, [bkt + laneoff], ones)
            return 0
        lax.fori_loop(0, _VECS, body, 0)

    nchunks = _ROWS_PER_W * _CHUNKS_PER_ROW
    pending = [None, None]
    pending[0] = start(0, 0)
    for r in range(nchunks):
        slot = r % 2
        if r + 1 < nchunks:
            pending[(r + 1) % 2] = start(r + 1, (r + 1) % 2)
        pending[slot].wait()
        process(bufs[slot])

    def mbody(c, _):
        acc = hist1d[pl.ds(c * 16, 16)]
        for l in range(1, 16):
            acc = acc + hist1d[pl.ds(l * _NBUCKETS + c * 16, 16)]
        merged[pl.ds(c * 16, 16)] = acc
        return 0
    lax.fori_loop(0, _NBUCKETS // 16, mbody, 0)

    pltpu.sync_copy(merged, shared.at[sid])
    plsc.subcore_barrier()

    @pl.when(sid == 0)
    def _reduce():
        for r in range(16):
            pltpu.sync_copy(
                shared.at[r], hist1d.at[pl.ds(r * _NBUCKETS, _NBUCKETS)]
            )
        lax.fori_loop(0, _NBUCKETS // 16, mbody, 0)
        pltpu.sync_copy(merged, out_hbm.at[cid])


_hist_sc = functools.partial(
    pl.kernel,
    out_type=jax.ShapeDtypeStruct((_NC, _NBUCKETS), jnp.int32),
    mesh=plsc.VectorSubcoreMesh(core_axis_name="c", subcore_axis_name="s"),
    compiler_params=pltpu.CompilerParams(needs_layout_passes=False),
    scratch_types=[
        pltpu.VMEM((_CHUNK,), jnp.int32),
        pltpu.VMEM((_CHUNK,), jnp.int32),
        pltpu.VMEM((16 * _NBUCKETS,), jnp.int32),
        pltpu.VMEM((_NBUCKETS,), jnp.int32),
        pltpu.VMEM_SHARED((16, _NBUCKETS), jnp.int32),
        pltpu.SemaphoreType.DMA,
        pltpu.SemaphoreType.DMA,
    ],
)(_hist_body)


def _apply_body(pre_ref, fine_ref, hist_ref, out_ref, thr):
    s = pl.program_id(0)

    @pl.when(s == 0)
    def _threshold():
        hr = hist_ref[...]
        hf = (hr[0] + hr[1]).astype(jnp.float32)
        ic = lax.broadcasted_iota(jnp.int32, (128, 128), 0)
        jc = lax.broadcasted_iota(jnp.int32, (128, 128), 1)
        u = (ic >= jc).astype(jnp.float32)
        row_suffix = lax.dot(hf, u, precision=lax.Precision.HIGHEST)
        totals = row_suffix[:, 0:1]
        ir = lax.broadcasted_iota(jnp.int32, (32, 32), 0)
        jr = lax.broadcasted_iota(jnp.int32, (32, 32), 1)
        g = (jr > ir).astype(jnp.float32)
        rstrict = lax.dot(g, totals, precision=lax.Precision.HIGHEST)
        sfx = rstrict + row_suffix
        bstar = jnp.sum((sfx >= jnp.float32(_K)).astype(jnp.int32)) - 1
        thr[0] = bstar * jnp.int32(1 << _SHIFT)

    @pl.when(s > 0)
    def _apply():
        pre = pre_ref[...]
        tv = fine_ref[...] - pre
        bits = lax.bitcast_convert_type(tv, jnp.int32)
        keep = jnp.bitwise_and(bits, jnp.int32(0x7FFFFFFF)) >= thr[0]
        out_ref[...] = pre + jnp.where(keep, tv, 0.0)


@jax.jit
def kernel(pretensor, finetensor):
    blk = pl.BlockSpec((_BLK, _C), lambda b: (b, 0))
    idx = pl.pallas_call(
        _stage_body,
        grid=(_NB,),
        in_specs=[blk, blk],
        out_specs=blk,
        out_shape=jax.ShapeDtypeStruct((_R, _C), jnp.int32),
    )(pretensor, finetensor)

    hist = _hist_sc(idx.reshape(_NROWS, _SCROW))
    hist3 = hist.reshape(_NC, 32, 128)

    data_spec = pl.BlockSpec(
        (_BLK, _C), lambda s: (jnp.maximum(s - 1, 0), 0)
    )
    return pl.pallas_call(
        _apply_body,
        grid=(_NB + 1,),
        in_specs=[
            data_spec,
            data_spec,
            pl.BlockSpec((_NC, 32, 128), lambda s: (0, 0, 0)),
        ],
        out_specs=data_spec,
        out_shape=jax.ShapeDtypeStruct((_R, _C), jnp.float32),
        scratch_shapes=[pltpu.SMEM((1,), jnp.int32)],
    )(pretensor, finetensor, hist3)


# speedup vs baseline: 1.0809x; 1.0494x over previous
import functools

import jax
import jax.numpy as jnp
from jax import lax
from jax.experimental import pallas as pl
from jax.experimental.pallas import tpu as pltpu
from jax.experimental.pallas import tpu_sc as plsc

_R, _C = 2048, 8192
_N = _R * _C
_K = int(0.05 * _N)
_NBUCKETS = 4096
_SHIFT = 19

_NC, _NS = 2, 16
_NW = _NC * _NS
_SCROW = 32768
_NROWS = _N // _SCROW
_ROWS_PER_W = _NROWS // _NW
_CHUNK = 8192
_CHUNKS_PER_ROW = _SCROW // _CHUNK
_VECS = _CHUNK // 16

_BLK = 128
_NB = _R // _BLK


def _stage_body(pre_ref, fine_ref, idx_ref):
    tv = fine_ref[...] - pre_ref[...]
    bits = lax.bitcast_convert_type(tv, jnp.int32)
    idx_ref[...] = lax.shift_right_logical(
        jnp.bitwise_and(bits, jnp.int32(0x7FFFFFFF)), _SHIFT
    )


def _hist_body(idx_hbm, out_hbm, b0, b1, hist1d, merged, shared, s0, s1):
    cid = lax.axis_index("c")
    sid = lax.axis_index("s")
    wid = sid * _NC + cid
    base_row = wid * _ROWS_PER_W

    laneoff = lax.iota(jnp.int32, 16) * _NBUCKETS
    ones = jnp.ones((16,), jnp.int32)
    zeros = jnp.zeros((16,), jnp.int32)

    def zbody(c, _):
        for u in range(8):
            hist1d[pl.ds(c * 128 + u * 16, 16)] = zeros
        return 0
    lax.fori_loop(0, 16 * _NBUCKETS // 128, zbody, 0)

    bufs = [b0, b1]
    sems = [s0, s1]

    def start(r, slot):
        row = base_row + (r // _CHUNKS_PER_ROW)
        off = (r % _CHUNKS_PER_ROW) * _CHUNK
        return pltpu.async_copy(
            idx_hbm.at[row, pl.ds(off, _CHUNK)], bufs[slot], sems[slot]
        )

    def process(buf_ref):
        def body(i, _):
            for u in range(8):
                bkt = buf_ref[pl.ds(i * 128 + u * 16, 16)]
                plsc.addupdate_scatter(hist1d, [bkt + laneoff], ones)
            return 0
        lax.fori_loop(0, _VECS // 8, body, 0)

    nchunks = _ROWS_PER_W * _CHUNKS_PER_ROW
    pending = [None, None]
    pending[0] = start(0, 0)
    for r in range(nchunks):
        slot = r % 2
        if r + 1 < nchunks:
            pending[(r + 1) % 2] = start(r + 1, (r + 1) % 2)
        pending[slot].wait()
        process(bufs[slot])

    def mbody(c, _):
        acc = hist1d[pl.ds(c * 16, 16)]
        for l in range(1, 16):
            acc = acc + hist1d[pl.ds(l * _NBUCKETS + c * 16, 16)]
        merged[pl.ds(c * 16, 16)] = acc
        return 0
    lax.fori_loop(0, _NBUCKETS // 16, mbody, 0)

    pltpu.sync_copy(merged, shared.at[sid])
    plsc.subcore_barrier()

    @pl.when(sid == 0)
    def _reduce():
        for r in range(16):
            pltpu.sync_copy(
                shared.at[r], hist1d.at[pl.ds(r * _NBUCKETS, _NBUCKETS)]
            )
        lax.fori_loop(0, _NBUCKETS // 16, mbody, 0)
        pltpu.sync_copy(merged, out_hbm.at[cid])


_hist_sc = functools.partial(
    pl.kernel,
    out_type=jax.ShapeDtypeStruct((_NC, _NBUCKETS), jnp.int32),
    mesh=plsc.VectorSubcoreMesh(core_axis_name="c", subcore_axis_name="s"),
    compiler_params=pltpu.CompilerParams(needs_layout_passes=False),
    scratch_types=[
        pltpu.VMEM((_CHUNK,), jnp.int32),
        pltpu.VMEM((_CHUNK,), jnp.int32),
        pltpu.VMEM((16 * _NBUCKETS,), jnp.int32),
        pltpu.VMEM((_NBUCKETS,), jnp.int32),
        pltpu.VMEM_SHARED((16, _NBUCKETS), jnp.int32),
        pltpu.SemaphoreType.DMA,
        pltpu.SemaphoreType.DMA,
    ],
)(_hist_body)


def _apply_body(pre_ref, fine_ref, hist_ref, out_ref, thr):
    s = pl.program_id(0)

    @pl.when(s == 0)
    def _threshold():
        hr = hist_ref[...]
        hf = (hr[0] + hr[1]).astype(jnp.float32)
        ic = lax.broadcasted_iota(jnp.int32, (128, 128), 0)
        jc = lax.broadcasted_iota(jnp.int32, (128, 128), 1)
        u = (ic >= jc).astype(jnp.float32)
        row_suffix = lax.dot(hf, u, precision=lax.Precision.HIGHEST)
        totals = row_suffix[:, 0:1]
        ir = lax.broadcasted_iota(jnp.int32, (32, 32), 0)
        jr = lax.broadcasted_iota(jnp.int32, (32, 32), 1)
        g = (jr > ir).astype(jnp.float32)
        rstrict = lax.dot(g, totals, precision=lax.Precision.HIGHEST)
        sfx = rstrict + row_suffix
        bstar = jnp.sum((sfx >= jnp.float32(_K)).astype(jnp.int32)) - 1
        thr[0] = bstar * jnp.int32(1 << _SHIFT)

    @pl.when(s > 0)
    def _apply():
        pre = pre_ref[...]
        tv = fine_ref[...] - pre
        bits = lax.bitcast_convert_type(tv, jnp.int32)
        keep = jnp.bitwise_and(bits, jnp.int32(0x7FFFFFFF)) >= thr[0]
        out_ref[...] = pre + jnp.where(keep, tv, 0.0)


@jax.jit
def kernel(pretensor, finetensor):
    blk = pl.BlockSpec((_BLK, _C), lambda b: (b, 0))
    idx = pl.pallas_call(
        _stage_body,
        grid=(_NB,),
        in_specs=[blk, blk],
        out_specs=blk,
        out_shape=jax.ShapeDtypeStruct((_R, _C), jnp.int32),
    )(pretensor, finetensor)

    hist = _hist_sc(idx.reshape(_NROWS, _SCROW))
    hist3 = hist.reshape(_NC, 32, 128)

    data_spec = pl.BlockSpec(
        (_BLK, _C), lambda s: (jnp.maximum(s - 1, 0), 0)
    )
    return pl.pallas_call(
        _apply_body,
        grid=(_NB + 1,),
        in_specs=[
            data_spec,
            data_spec,
            pl.BlockSpec((_NC, 32, 128), lambda s: (0, 0, 0)),
        ],
        out_specs=data_spec,
        out_shape=jax.ShapeDtypeStruct((_R, _C), jnp.float32),
        scratch_shapes=[pltpu.SMEM((1,), jnp.int32)],
    )(pretensor, finetensor, hist3)
